# Initial kernel scaffold; baseline (speedup 1.0000x reference)
import jax
import jax.numpy as jnp
from jax.experimental import pallas as pl


def kernel(edge_index, item_features, U, I, Wf, bf, W1, b1, W2, b2, W3, b3):
    raise NotImplementedError("write your pallas kernel here")


# speedup vs baseline: 2.8320x; 2.8320x over previous
import functools

import jax
import jax.numpy as jnp
from jax import lax
from jax.experimental import pallas as pl
from jax.experimental.pallas import tpu as pltpu
from jax.experimental.pallas import tpu_sc as plsc

NU = 25000
NI = 25000
NN = NU + NI
D = 64
KG = 128
E = 800000
EP = 802816
EROWS = EP // 128
TRASH = NN

SUB_ROWS = EROWS // 16
SC_BLK = 1
SC_OUTER = SUB_ROWS // SC_BLK
AROWS = 51200
AZSPAN = AROWS // 16

DEG_ROWS = EROWS // 32
DEG_OUTER = DEG_ROWS // 4
NPASS = 8
PASS_N = 6400
DROWS = 6528
DSPAN = DROWS // 16

R = 1000
GRID = NN // R
UBLK = NU // R
RD = 200
DGRID = NN // RD

_MESH = dict(core_axis_name="c", subcore_axis_name="s")
_f32 = jnp.float32


def _fill(ref, rows, value):
    zero16 = jnp.full((16,), value, _f32)

    def body(i, carry):
        for k in range(ref.shape[1] // 16):
            ref[i, pl.ds(k * 16, 16)] = zero16
        return carry

    lax.fori_loop(0, rows, body, 0)


def _deg_body(dstb, out, idx_d, idx_l, ones_v, zbuf, dbuf, dacc, sem):
    c = lax.axis_index("c")
    s = lax.axis_index("s")
    wid = s * 2 + c
    _fill(ones_v, 128, 1.0)
    _fill(zbuf, 128, 0.0)
    for p in range(NPASS):
        for r0, rn in ((0, 128), (128, 128), (256, 128), (384, 24)):
            pltpu.sync_copy(zbuf.at[pl.ds(0, rn)],
                            dacc.at[pl.ds(s * DSPAN + r0, rn)])
        plsc.subcore_barrier()

        def outer(i, carry):
            base = wid * DEG_ROWS + i * 4
            pltpu.sync_copy(dstb.at[pl.ds(base, 4)], idx_d)
            for r in range(4):
                for k in range(8):
                    v = idx_d[r, pl.ds(k * 16, 16)]
                    l = v - p * PASS_N
                    bad = (l < 0) | (l >= PASS_N)
                    idx_l[r, pl.ds(k * 16, 16)] = jnp.where(bad, PASS_N, l)
            for j in range(4):
                pltpu.sync_copy(ones_v, dacc.at[idx_l.at[j]], add=True)
            return carry

        lax.fori_loop(0, DEG_OUTER, outer, 0)
        plsc.subcore_barrier()
        for r0, rn in ((0, 128), (128, 128), (256, 128), (384, 24)):
            pltpu.sync_copy(dacc.at[pl.ds(s * DSPAN + r0, rn)],
                            dbuf.at[pl.ds(0, rn)])
            pltpu.sync_copy(
                dbuf.at[pl.ds(0, rn)],
                out.at[pl.ds((c * NPASS + p) * DROWS + s * DSPAN + r0, rn)])
        plsc.subcore_barrier()


@functools.partial(
    pl.kernel,
    out_type=jax.ShapeDtypeStruct((2 * NPASS * DROWS, 128), _f32),
    mesh=plsc.VectorSubcoreMesh(**_MESH),
    compiler_params=pltpu.CompilerParams(use_tc_tiling_on_sc=False),
    scratch_types=[
        pltpu.VMEM((4, 128), jnp.int32),
        pltpu.VMEM((4, 128), jnp.int32),
        pltpu.VMEM((128, 128), _f32),
        pltpu.VMEM((128, 128), _f32),
        pltpu.VMEM((128, 128), _f32),
        pltpu.VMEM_SHARED((DROWS, 128), _f32),
        pltpu.SemaphoreType.DMA,
    ],
)
def _sc_deg(dstb, out, idx_d, idx_l, ones_v, zbuf, dbuf, dacc, sem):
    _deg_body(dstb, out, idx_d, idx_l, ones_v, zbuf, dbuf, dacc, sem)


def _scatter_body(srcb, dstb, y, out, idx_s, idx_d, rows128, rows32, zbuf,
                  acc, sem):
    c = lax.axis_index("c")
    s = lax.axis_index("s")
    _fill(zbuf, 128, 0.0)
    for k in range(AZSPAN // 128):
        pltpu.sync_copy(zbuf, acc.at[pl.ds(s * AZSPAN + k * 128, 128)])
    plsc.subcore_barrier()

    def outer(i, carry):
        base = s * SUB_ROWS + i * SC_BLK
        pltpu.sync_copy(srcb.at[pl.ds(base, SC_BLK)], idx_s)
        pltpu.sync_copy(dstb.at[pl.ds(base, SC_BLK)], idx_d)
        descs = [
            pltpu.async_copy(y.at[idx_s.at[j]],
                             rows128.at[pl.ds(j * 128, 128)], sem)
            for j in range(SC_BLK)
        ]
        for de in descs:
            de.wait()
        coff = c * 32

        def extract(r, carry):
            rows32[r, pl.ds(0, 16)] = rows128[r, pl.ds(coff, 16)]
            rows32[r, pl.ds(16, 16)] = rows128[r, pl.ds(coff + 16, 16)]
            return carry

        lax.fori_loop(0, SC_BLK * 128, extract, 0)
        for j in range(SC_BLK):
            pltpu.sync_copy(rows32.at[pl.ds(j * 128, 128)],
                            acc.at[idx_d.at[j]], add=True)
        return carry

    lax.fori_loop(0, SC_OUTER, outer, 0)
    plsc.subcore_barrier()
    for k in range(25):
        r0 = s * AZSPAN + k * 128
        pltpu.sync_copy(acc.at[pl.ds(r0, 128)], rows32)
        pltpu.sync_copy(rows32,
                        out.at[c, pl.ds(r0, 128), pl.ds(0, 32)])


@functools.partial(
    pl.kernel,
    out_type=jax.ShapeDtypeStruct((2, AROWS, 128), _f32),
    mesh=plsc.VectorSubcoreMesh(**_MESH),
    compiler_params=pltpu.CompilerParams(use_tc_tiling_on_sc=False),
    scratch_types=[
        pltpu.VMEM((SC_BLK, 128), jnp.int32),
        pltpu.VMEM((SC_BLK, 128), jnp.int32),
        pltpu.VMEM((SC_BLK * 128, 128), _f32),
        pltpu.VMEM((SC_BLK * 128, 32), _f32),
        pltpu.VMEM((128, 32), _f32),
        pltpu.VMEM_SHARED((AROWS, 32), _f32),
        pltpu.SemaphoreType.DMA,
    ],
)
def _sc_scatter(srcb, dstb, y, out, idx_s, idx_d, rows128, rows32, zbuf, acc,
                sem):
    _scatter_body(srcb, dstb, y, out, idx_s, idx_d, rows128, rows32, zbuf,
                  acc, sem)


def _dis_body(deg_ref, o_ref):
    dg = deg_ref[0, 0, :, 0:1] + deg_ref[1, 0, :, 0:1]
    o_ref[...] = jnp.where(dg > 0, lax.rsqrt(jnp.maximum(dg, 1e-12)), 0.0)


def _tc_dis(deg2):
    return pl.pallas_call(
        _dis_body,
        grid=(DGRID,),
        in_specs=[
            pl.BlockSpec((2, 1, RD, 128),
                         lambda i: (0, i // (PASS_N // RD),
                                    i % (PASS_N // RD), 0)),
        ],
        out_specs=pl.BlockSpec((RD, 1), lambda i: (i, 0)),
        out_shape=jax.ShapeDtypeStruct((NN, 1), _f32),
    )(deg2)


def _dis_block(deg_ref):
    dg = deg_ref[0, 0, :, 0:1] + deg_ref[1, 0, :, 0:1]
    return jnp.where(dg > 0, lax.rsqrt(jnp.maximum(dg, 1e-12)), 0.0)


def _store_y(y, y_ref):
    y_ref[:, 0:D] = y
    y_ref[:, D:128] = jnp.zeros((y.shape[0], 128 - D), _f32)


def _tc_first_body(u_ref, i_ref, f_ref, wft_ref, bf_ref, w1_ref, s_ref,
                   x0_ref, y_ref):
    i = pl.program_id(0)
    sc = s_ref[...]
    item = i_ref[...] + jnp.dot(f_ref[...], wft_ref[...],
                                preferred_element_type=_f32) + bf_ref[...]
    x0 = jnp.where(i < UBLK, u_ref[...], item)
    x0_ref[...] = x0
    _store_y(jnp.dot(x0, w1_ref[...], preferred_element_type=_f32) * sc,
             y_ref)


def _tc_first(U, I, feat, WfT, bf2, W1, dis):
    return pl.pallas_call(
        _tc_first_body,
        grid=(GRID,),
        in_specs=[
            pl.BlockSpec((R, D), lambda i: (jnp.minimum(i, UBLK - 1), 0)),
            pl.BlockSpec((R, D), lambda i: (jnp.maximum(i - UBLK, 0), 0)),
            pl.BlockSpec((R, KG), lambda i: (jnp.maximum(i - UBLK, 0), 0)),
            pl.BlockSpec((KG, D), lambda i: (0, 0)),
            pl.BlockSpec((1, D), lambda i: (0, 0)),
            pl.BlockSpec((D, D), lambda i: (0, 0)),
            pl.BlockSpec((R, 1), lambda i: (i, 0)),
        ],
        out_specs=[
            pl.BlockSpec((R, D), lambda i: (i, 0)),
            pl.BlockSpec((R, 128), lambda i: (i, 0)),
        ],
        out_shape=[
            jax.ShapeDtypeStruct((NN, D), _f32),
            jax.ShapeDtypeStruct((NN, 128), _f32),
        ],
    )(U, I, feat, WfT, bf2, W1, dis)


def _merge_acc(acc_ref):
    return jnp.concatenate([acc_ref[0, :, 0:32], acc_ref[1, :, 0:32]],
                           axis=1)


def _tc_mid_body(acc_ref, s_ref, b_ref, w_ref, x_ref, y_ref):
    sc = s_ref[...]
    x = _merge_acc(acc_ref) * sc + b_ref[...]
    x_ref[...] = x
    _store_y(jnp.dot(x, w_ref[...], preferred_element_type=_f32) * sc, y_ref)


def _tc_mid(acc, dis, b2, W):
    return pl.pallas_call(
        _tc_mid_body,
        grid=(GRID,),
        in_specs=[
            pl.BlockSpec((2, R, 128), lambda i: (0, i, 0)),
            pl.BlockSpec((R, 1), lambda i: (i, 0)),
            pl.BlockSpec((1, D), lambda i: (0, 0)),
            pl.BlockSpec((D, D), lambda i: (0, 0)),
        ],
        out_specs=[
            pl.BlockSpec((R, D), lambda i: (i, 0)),
            pl.BlockSpec((R, 128), lambda i: (i, 0)),
        ],
        out_shape=[
            jax.ShapeDtypeStruct((NN, D), _f32),
            jax.ShapeDtypeStruct((NN, 128), _f32),
        ],
    )(acc, dis, b2, W)


def _tc_last_body(acc_ref, s_ref, b_ref, x0_ref, x1_ref, x2_ref, o_ref):
    sc = s_ref[...]
    x3 = _merge_acc(acc_ref) * sc + b_ref[...]
    o_ref[...] = (x0_ref[...] + x1_ref[...] + x2_ref[...] + x3) * 0.25


def _tc_last(acc, dis, b2, x0, x1, x2):
    blk = pl.BlockSpec((R, D), lambda i: (i, 0))
    return pl.pallas_call(
        _tc_last_body,
        grid=(GRID,),
        in_specs=[
            pl.BlockSpec((2, R, 128), lambda i: (0, i, 0)),
            pl.BlockSpec((R, 1), lambda i: (i, 0)),
            pl.BlockSpec((1, D), lambda i: (0, 0)),
            blk, blk, blk,
        ],
        out_specs=blk,
        out_shape=jax.ShapeDtypeStruct((NN, D), _f32),
    )(acc, dis, b2, x0, x1, x2)


def kernel(edge_index, item_features, U, I, Wf, bf, W1, b1, W2, b2, W3, b3):
    src = edge_index[0].astype(jnp.int32)
    dst = edge_index[1].astype(jnp.int32)
    pad = EP - E
    srcb = jnp.concatenate([src, jnp.zeros((pad,), jnp.int32)])
    dstb = jnp.concatenate([dst, jnp.full((pad,), TRASH, jnp.int32)])
    srcb = srcb.reshape(EROWS, 128)
    dstb = dstb.reshape(EROWS, 128)
    WfT = Wf.T
    bf2 = bf.reshape(1, D)

    deg2 = _sc_deg(dstb).reshape(2, NPASS, DROWS, 128)
    dis = _tc_dis(deg2)
    x0, y = _tc_first(U, I, item_features, WfT, bf2, W1, dis)
    acc = _sc_scatter(srcb, dstb, y)
    x1, y = _tc_mid(acc, dis, b1.reshape(1, D), W2)
    acc = _sc_scatter(srcb, dstb, y)
    x2, y = _tc_mid(acc, dis, b2.reshape(1, D), W3)
    acc = _sc_scatter(srcb, dstb, y)
    out = _tc_last(acc, dis, b3.reshape(1, D), x0, x1, x2)
    return out[:NU], out[NU:]


# speedup vs baseline: 3.1010x; 1.0950x over previous
import functools

import jax
import jax.numpy as jnp
from jax import lax
from jax.experimental import pallas as pl
from jax.experimental.pallas import tpu as pltpu
from jax.experimental.pallas import tpu_sc as plsc

NU = 25000
NI = 25000
NN = NU + NI
D = 64
KG = 128
E = 800000
EP = 802816
EROWS = EP // 128
TRASH = NN

SUB_ROWS = EROWS // 16
SC_BLK = 1
SC_OUTER = SUB_ROWS // SC_BLK
AROWS = 51200
AZSPAN = AROWS // 16

DEG_ROWS = EROWS // 32
DEG_OUTER = DEG_ROWS // 4
NPASS = 8
PASS_N = 6400
DROWS = 6528
DSPAN = DROWS // 16

R = 1000
GRID = NN // R
UBLK = NU // R
RD = 200
DGRID = NN // RD

_MESH = dict(core_axis_name="c", subcore_axis_name="s")
_f32 = jnp.float32


def _fill(ref, rows, value):
    zero16 = jnp.full((16,), value, _f32)

    def body(i, carry):
        for k in range(ref.shape[1] // 16):
            ref[i, pl.ds(k * 16, 16)] = zero16
        return carry

    lax.fori_loop(0, rows, body, 0)


def _deg_body(dstb, out, idx_d, idx_l, ones_v, zbuf, dbuf, dacc, sem):
    c = lax.axis_index("c")
    s = lax.axis_index("s")
    wid = s * 2 + c
    _fill(ones_v, 128, 1.0)
    _fill(zbuf, 128, 0.0)
    for p in range(NPASS):
        for r0, rn in ((0, 128), (128, 128), (256, 128), (384, 24)):
            pltpu.sync_copy(zbuf.at[pl.ds(0, rn)],
                            dacc.at[pl.ds(s * DSPAN + r0, rn)])
        plsc.subcore_barrier()

        def outer(i, carry):
            base = wid * DEG_ROWS + i * 4
            pltpu.sync_copy(dstb.at[pl.ds(base, 4)], idx_d)
            for r in range(4):
                for k in range(8):
                    v = idx_d[r, pl.ds(k * 16, 16)]
                    l = v - p * PASS_N
                    bad = (l < 0) | (l >= PASS_N)
                    idx_l[r, pl.ds(k * 16, 16)] = jnp.where(bad, PASS_N, l)
            for j in range(4):
                pltpu.sync_copy(ones_v, dacc.at[idx_l.at[j]], add=True)
            return carry

        lax.fori_loop(0, DEG_OUTER, outer, 0)
        plsc.subcore_barrier()
        for r0, rn in ((0, 128), (128, 128), (256, 128), (384, 24)):
            pltpu.sync_copy(dacc.at[pl.ds(s * DSPAN + r0, rn)],
                            dbuf.at[pl.ds(0, rn)])
            pltpu.sync_copy(
                dbuf.at[pl.ds(0, rn)],
                out.at[pl.ds((c * NPASS + p) * DROWS + s * DSPAN + r0, rn)])
        plsc.subcore_barrier()


@functools.partial(
    pl.kernel,
    out_type=jax.ShapeDtypeStruct((2 * NPASS * DROWS, 128), _f32),
    mesh=plsc.VectorSubcoreMesh(**_MESH),
    compiler_params=pltpu.CompilerParams(use_tc_tiling_on_sc=False),
    scratch_types=[
        pltpu.VMEM((4, 128), jnp.int32),
        pltpu.VMEM((4, 128), jnp.int32),
        pltpu.VMEM((128, 128), _f32),
        pltpu.VMEM((128, 128), _f32),
        pltpu.VMEM((128, 128), _f32),
        pltpu.VMEM_SHARED((DROWS, 128), _f32),
        pltpu.SemaphoreType.DMA,
    ],
)
def _sc_deg(dstb, out, idx_d, idx_l, ones_v, zbuf, dbuf, dacc, sem):
    _deg_body(dstb, out, idx_d, idx_l, ones_v, zbuf, dbuf, dacc, sem)


def _scatter_body(srcb, dstb, y, out, idx_s, idx_d, rows128, rows32, zbuf,
                  acc, sem):
    c = lax.axis_index("c")
    s = lax.axis_index("s")
    _fill(zbuf, 128, 0.0)
    for k in range(AZSPAN // 128):
        pltpu.sync_copy(zbuf, acc.at[pl.ds(s * AZSPAN + k * 128, 128)])
    plsc.subcore_barrier()

    coff = c * 32
    base0 = s * SUB_ROWS
    pltpu.sync_copy(srcb.at[pl.ds(base0, 1)], idx_s)
    pltpu.sync_copy(dstb.at[pl.ds(base0, 1)], idx_d)
    pltpu.async_copy(y.at[idx_s.at[0]], rows128, sem)

    def outer(i, carry):
        pltpu.make_async_copy(y.at[idx_s.at[0]], rows128, sem).wait()

        def extract(r, carry):
            rows32[r, pl.ds(0, 16)] = rows128[r, pl.ds(coff, 16)]
            rows32[r, pl.ds(16, 16)] = rows128[r, pl.ds(coff + 16, 16)]
            return carry

        lax.fori_loop(0, 128, extract, 0)

        @pl.when(i < SC_OUTER - 1)
        def _fire_next():
            pltpu.sync_copy(srcb.at[pl.ds(base0 + i + 1, 1)], idx_s)
            pltpu.async_copy(y.at[idx_s.at[0]], rows128, sem)

        pltpu.sync_copy(rows32, acc.at[idx_d.at[0]], add=True)

        @pl.when(i < SC_OUTER - 1)
        def _stage_dst():
            pltpu.sync_copy(dstb.at[pl.ds(base0 + i + 1, 1)], idx_d)

        return carry

    lax.fori_loop(0, SC_OUTER, outer, 0)
    plsc.subcore_barrier()
    for k in range(25):
        r0 = s * AZSPAN + k * 128
        pltpu.sync_copy(acc.at[pl.ds(r0, 128)], rows32)
        pltpu.sync_copy(rows32,
                        out.at[c, pl.ds(r0, 128), pl.ds(0, 32)])


@functools.partial(
    pl.kernel,
    out_type=jax.ShapeDtypeStruct((2, AROWS, 128), _f32),
    mesh=plsc.VectorSubcoreMesh(**_MESH),
    compiler_params=pltpu.CompilerParams(use_tc_tiling_on_sc=False),
    scratch_types=[
        pltpu.VMEM((SC_BLK, 128), jnp.int32),
        pltpu.VMEM((SC_BLK, 128), jnp.int32),
        pltpu.VMEM((SC_BLK * 128, 128), _f32),
        pltpu.VMEM((SC_BLK * 128, 32), _f32),
        pltpu.VMEM((128, 32), _f32),
        pltpu.VMEM_SHARED((AROWS, 32), _f32),
        pltpu.SemaphoreType.DMA,
    ],
)
def _sc_scatter(srcb, dstb, y, out, idx_s, idx_d, rows128, rows32, zbuf, acc,
                sem):
    _scatter_body(srcb, dstb, y, out, idx_s, idx_d, rows128, rows32, zbuf,
                  acc, sem)


def _dis_body(deg_ref, o_ref):
    dg = deg_ref[0, 0, :, 0:1] + deg_ref[1, 0, :, 0:1]
    o_ref[...] = jnp.where(dg > 0, lax.rsqrt(jnp.maximum(dg, 1e-12)), 0.0)


def _tc_dis(deg2):
    return pl.pallas_call(
        _dis_body,
        grid=(DGRID,),
        in_specs=[
            pl.BlockSpec((2, 1, RD, 128),
                         lambda i: (0, i // (PASS_N // RD),
                                    i % (PASS_N // RD), 0)),
        ],
        out_specs=pl.BlockSpec((RD, 1), lambda i: (i, 0)),
        out_shape=jax.ShapeDtypeStruct((NN, 1), _f32),
    )(deg2)


def _dis_block(deg_ref):
    dg = deg_ref[0, 0, :, 0:1] + deg_ref[1, 0, :, 0:1]
    return jnp.where(dg > 0, lax.rsqrt(jnp.maximum(dg, 1e-12)), 0.0)


def _store_y(y, y_ref):
    y_ref[:, 0:D] = y
    y_ref[:, D:128] = jnp.zeros((y.shape[0], 128 - D), _f32)


def _tc_first_body(u_ref, i_ref, f_ref, wft_ref, bf_ref, w1_ref, s_ref,
                   x0_ref, y_ref):
    i = pl.program_id(0)
    sc = s_ref[...]
    item = i_ref[...] + jnp.dot(f_ref[...], wft_ref[...],
                                preferred_element_type=_f32) + bf_ref[...]
    x0 = jnp.where(i < UBLK, u_ref[...], item)
    x0_ref[...] = x0
    _store_y(jnp.dot(x0, w1_ref[...], preferred_element_type=_f32) * sc,
             y_ref)


def _tc_first(U, I, feat, WfT, bf2, W1, dis):
    return pl.pallas_call(
        _tc_first_body,
        grid=(GRID,),
        in_specs=[
            pl.BlockSpec((R, D), lambda i: (jnp.minimum(i, UBLK - 1), 0)),
            pl.BlockSpec((R, D), lambda i: (jnp.maximum(i - UBLK, 0), 0)),
            pl.BlockSpec((R, KG), lambda i: (jnp.maximum(i - UBLK, 0), 0)),
            pl.BlockSpec((KG, D), lambda i: (0, 0)),
            pl.BlockSpec((1, D), lambda i: (0, 0)),
            pl.BlockSpec((D, D), lambda i: (0, 0)),
            pl.BlockSpec((R, 1), lambda i: (i, 0)),
        ],
        out_specs=[
            pl.BlockSpec((R, D), lambda i: (i, 0)),
            pl.BlockSpec((R, 128), lambda i: (i, 0)),
        ],
        out_shape=[
            jax.ShapeDtypeStruct((NN, D), _f32),
            jax.ShapeDtypeStruct((NN, 128), _f32),
        ],
    )(U, I, feat, WfT, bf2, W1, dis)


def _merge_acc(acc_ref):
    return jnp.concatenate([acc_ref[0, :, 0:32], acc_ref[1, :, 0:32]],
                           axis=1)


def _tc_mid_body(acc_ref, s_ref, b_ref, w_ref, x_ref, y_ref):
    sc = s_ref[...]
    x = _merge_acc(acc_ref) * sc + b_ref[...]
    x_ref[...] = x
    _store_y(jnp.dot(x, w_ref[...], preferred_element_type=_f32) * sc, y_ref)


def _tc_mid(acc, dis, b2, W):
    return pl.pallas_call(
        _tc_mid_body,
        grid=(GRID,),
        in_specs=[
            pl.BlockSpec((2, R, 128), lambda i: (0, i, 0)),
            pl.BlockSpec((R, 1), lambda i: (i, 0)),
            pl.BlockSpec((1, D), lambda i: (0, 0)),
            pl.BlockSpec((D, D), lambda i: (0, 0)),
        ],
        out_specs=[
            pl.BlockSpec((R, D), lambda i: (i, 0)),
            pl.BlockSpec((R, 128), lambda i: (i, 0)),
        ],
        out_shape=[
            jax.ShapeDtypeStruct((NN, D), _f32),
            jax.ShapeDtypeStruct((NN, 128), _f32),
        ],
    )(acc, dis, b2, W)


def _tc_last_body(acc_ref, s_ref, b_ref, x0_ref, x1_ref, x2_ref, o_ref):
    sc = s_ref[...]
    x3 = _merge_acc(acc_ref) * sc + b_ref[...]
    o_ref[...] = (x0_ref[...] + x1_ref[...] + x2_ref[...] + x3) * 0.25


def _tc_last(acc, dis, b2, x0, x1, x2):
    blk = pl.BlockSpec((R, D), lambda i: (i, 0))
    return pl.pallas_call(
        _tc_last_body,
        grid=(GRID,),
        in_specs=[
            pl.BlockSpec((2, R, 128), lambda i: (0, i, 0)),
            pl.BlockSpec((R, 1), lambda i: (i, 0)),
            pl.BlockSpec((1, D), lambda i: (0, 0)),
            blk, blk, blk,
        ],
        out_specs=blk,
        out_shape=jax.ShapeDtypeStruct((NN, D), _f32),
    )(acc, dis, b2, x0, x1, x2)


def kernel(edge_index, item_features, U, I, Wf, bf, W1, b1, W2, b2, W3, b3):
    src = edge_index[0].astype(jnp.int32)
    dst = edge_index[1].astype(jnp.int32)
    pad = EP - E
    srcb = jnp.concatenate([src, jnp.zeros((pad,), jnp.int32)])
    dstb = jnp.concatenate([dst, jnp.full((pad,), TRASH, jnp.int32)])
    srcb = srcb.reshape(EROWS, 128)
    dstb = dstb.reshape(EROWS, 128)
    WfT = Wf.T
    bf2 = bf.reshape(1, D)

    deg2 = _sc_deg(dstb).reshape(2, NPASS, DROWS, 128)
    dis = _tc_dis(deg2)
    x0, y = _tc_first(U, I, item_features, WfT, bf2, W1, dis)
    acc = _sc_scatter(srcb, dstb, y)
    x1, y = _tc_mid(acc, dis, b1.reshape(1, D), W2)
    acc = _sc_scatter(srcb, dstb, y)
    x2, y = _tc_mid(acc, dis, b2.reshape(1, D), W3)
    acc = _sc_scatter(srcb, dstb, y)
    out = _tc_last(acc, dis, b3.reshape(1, D), x0, x1, x2)
    return out[:NU], out[NU:]


# speedup vs baseline: 3.1553x; 1.0175x over previous
import functools

import jax
import jax.numpy as jnp
from jax import lax
from jax.experimental import pallas as pl
from jax.experimental.pallas import tpu as pltpu
from jax.experimental.pallas import tpu_sc as plsc

NU = 25000
NI = 25000
NN = NU + NI
D = 64
KG = 128
E = 800000
EP = 802816
EROWS = EP // 128
TRASH = NN

SUB_ROWS = EROWS // 16
SC_BLK = 1
SC_OUTER = SUB_ROWS // SC_BLK
AROWS = 51200
AZSPAN = AROWS // 16

DEG_ROWS = EROWS // 32
DEG_OUTER = DEG_ROWS // 4
NPASS = 8
PASS_N = 6400
DROWS = 6528
DSPAN = DROWS // 16

R = 1000
GRID = NN // R
UBLK = NU // R
RD = 200
DGRID = NN // RD

_MESH = dict(core_axis_name="c", subcore_axis_name="s")
_f32 = jnp.float32


def _fill(ref, rows, value):
    zero16 = jnp.full((16,), value, _f32)

    def body(i, carry):
        for k in range(ref.shape[1] // 16):
            ref[i, pl.ds(k * 16, 16)] = zero16
        return carry

    lax.fori_loop(0, rows, body, 0)


def _deg_body(dstb, out, idx_d, idx_l, ones_v, zbuf, dbuf, dacc, sem):
    c = lax.axis_index("c")
    s = lax.axis_index("s")
    wid = s * 2 + c
    _fill(ones_v, 128, 1.0)
    _fill(zbuf, 128, 0.0)
    for p in range(NPASS):
        for r0, rn in ((0, 128), (128, 128), (256, 128), (384, 24)):
            pltpu.sync_copy(zbuf.at[pl.ds(0, rn)],
                            dacc.at[pl.ds(s * DSPAN + r0, rn)])
        plsc.subcore_barrier()

        def outer(i, carry):
            base = wid * DEG_ROWS + i * 4
            pltpu.sync_copy(dstb.at[pl.ds(base, 4)], idx_d)
            for r in range(4):
                for k in range(8):
                    v = idx_d[r, pl.ds(k * 16, 16)]
                    l = v - p * PASS_N
                    bad = (l < 0) | (l >= PASS_N)
                    idx_l[r, pl.ds(k * 16, 16)] = jnp.where(bad, PASS_N, l)
            for j in range(4):
                pltpu.sync_copy(ones_v, dacc.at[idx_l.at[j]], add=True)
            return carry

        lax.fori_loop(0, DEG_OUTER, outer, 0)
        plsc.subcore_barrier()
        for r0, rn in ((0, 128), (128, 128), (256, 128), (384, 24)):
            pltpu.sync_copy(dacc.at[pl.ds(s * DSPAN + r0, rn)],
                            dbuf.at[pl.ds(0, rn)])
            pltpu.sync_copy(
                dbuf.at[pl.ds(0, rn)],
                out.at[pl.ds((c * NPASS + p) * DROWS + s * DSPAN + r0, rn)])
        plsc.subcore_barrier()


@functools.partial(
    pl.kernel,
    out_type=jax.ShapeDtypeStruct((2 * NPASS * DROWS, 128), _f32),
    mesh=plsc.VectorSubcoreMesh(**_MESH),
    compiler_params=pltpu.CompilerParams(use_tc_tiling_on_sc=False),
    scratch_types=[
        pltpu.VMEM((4, 128), jnp.int32),
        pltpu.VMEM((4, 128), jnp.int32),
        pltpu.VMEM((128, 128), _f32),
        pltpu.VMEM((128, 128), _f32),
        pltpu.VMEM((128, 128), _f32),
        pltpu.VMEM_SHARED((DROWS, 128), _f32),
        pltpu.SemaphoreType.DMA,
    ],
)
def _sc_deg(dstb, out, idx_d, idx_l, ones_v, zbuf, dbuf, dacc, sem):
    _deg_body(dstb, out, idx_d, idx_l, ones_v, zbuf, dbuf, dacc, sem)


def _scatter_body(srcb, dstb, y, out, idx_s1, idx_s2, idx_d1, idx_d2,
                  rowsa, rowsb, rows32, zbuf, acc, sema, semb):
    c = lax.axis_index("c")
    s = lax.axis_index("s")
    _fill(zbuf, 128, 0.0)
    for k in range(AZSPAN // 128):
        pltpu.sync_copy(zbuf, acc.at[pl.ds(s * AZSPAN + k * 128, 128)])
    plsc.subcore_barrier()

    base0 = s * SUB_ROWS

    def _extract(rows_bf):
        def body(r, carry):
            v = rows_bf[r, pl.ds(c * 32, 32)]
            a, b = plsc.unpack(v, format=plsc.PackFormat.INTERLEAVED,
                               preferred_element_type=_f32)
            rows32[r, pl.ds(0, 16)] = a
            rows32[r, pl.ds(16, 16)] = b
            return carry

        lax.fori_loop(0, 128, body, 0)

    pltpu.sync_copy(srcb.at[pl.ds(base0, 1)], idx_s1)
    pltpu.sync_copy(dstb.at[pl.ds(base0, 1)], idx_d1)
    pltpu.async_copy(y.at[idx_s1.at[0]], rowsa, sema)

    def outer(k, carry):
        pltpu.sync_copy(srcb.at[pl.ds(base0 + 2 * k + 1, 1)], idx_s2)
        pltpu.async_copy(y.at[idx_s2.at[0]], rowsb, semb)
        pltpu.make_async_copy(y.at[idx_s1.at[0]], rowsa, sema).wait()
        _extract(rowsa)
        pltpu.sync_copy(rows32, acc.at[idx_d1.at[0]], add=True)
        pltpu.sync_copy(dstb.at[pl.ds(base0 + 2 * k + 1, 1)], idx_d2)

        @pl.when(k < SC_OUTER // 2 - 1)
        def _fire_next():
            pltpu.sync_copy(srcb.at[pl.ds(base0 + 2 * k + 2, 1)], idx_s1)
            pltpu.async_copy(y.at[idx_s1.at[0]], rowsa, sema)

        pltpu.make_async_copy(y.at[idx_s2.at[0]], rowsb, semb).wait()
        _extract(rowsb)
        pltpu.sync_copy(rows32, acc.at[idx_d2.at[0]], add=True)

        @pl.when(k < SC_OUTER // 2 - 1)
        def _stage_dst():
            pltpu.sync_copy(dstb.at[pl.ds(base0 + 2 * k + 2, 1)], idx_d1)

        return carry

    lax.fori_loop(0, SC_OUTER // 2, outer, 0)
    plsc.subcore_barrier()
    for k in range(25):
        r0 = s * AZSPAN + k * 128
        pltpu.sync_copy(acc.at[pl.ds(r0, 128)], rows32)
        pltpu.sync_copy(rows32,
                        out.at[c, pl.ds(r0, 128), pl.ds(0, 32)])


@functools.partial(
    pl.kernel,
    out_type=jax.ShapeDtypeStruct((2, AROWS, 128), _f32),
    mesh=plsc.VectorSubcoreMesh(**_MESH),
    compiler_params=pltpu.CompilerParams(use_tc_tiling_on_sc=False,
                                         needs_layout_passes=False),
    scratch_types=[
        pltpu.VMEM((1, 128), jnp.int32),
        pltpu.VMEM((1, 128), jnp.int32),
        pltpu.VMEM((1, 128), jnp.int32),
        pltpu.VMEM((1, 128), jnp.int32),
        pltpu.VMEM((128, 128), jnp.bfloat16),
        pltpu.VMEM((128, 128), jnp.bfloat16),
        pltpu.VMEM((128, 32), _f32),
        pltpu.VMEM((128, 32), _f32),
        pltpu.VMEM_SHARED((AROWS, 32), _f32),
        pltpu.SemaphoreType.DMA,
        pltpu.SemaphoreType.DMA,
    ],
)
def _sc_scatter(srcb, dstb, y, out, idx_s1, idx_s2, idx_d1, idx_d2, rowsa,
                rowsb, rows32, zbuf, acc, sema, semb):
    _scatter_body(srcb, dstb, y, out, idx_s1, idx_s2, idx_d1, idx_d2, rowsa,
                  rowsb, rows32, zbuf, acc, sema, semb)


def _dis_body(deg_ref, o_ref):
    dg = deg_ref[0, 0, :, 0:1] + deg_ref[1, 0, :, 0:1]
    o_ref[...] = jnp.where(dg > 0, lax.rsqrt(jnp.maximum(dg, 1e-12)), 0.0)


def _tc_dis(deg2):
    return pl.pallas_call(
        _dis_body,
        grid=(DGRID,),
        in_specs=[
            pl.BlockSpec((2, 1, RD, 128),
                         lambda i: (0, i // (PASS_N // RD),
                                    i % (PASS_N // RD), 0)),
        ],
        out_specs=pl.BlockSpec((RD, 1), lambda i: (i, 0)),
        out_shape=jax.ShapeDtypeStruct((NN, 1), _f32),
    )(deg2)


def _dis_block(deg_ref):
    dg = deg_ref[0, 0, :, 0:1] + deg_ref[1, 0, :, 0:1]
    return jnp.where(dg > 0, lax.rsqrt(jnp.maximum(dg, 1e-12)), 0.0)


def _store_y(y, y_ref):
    r = y.shape[0]
    packed = jnp.transpose(y.reshape(r, 2, 2, 16), (0, 1, 3, 2)).reshape(r, D)
    y_ref[:, 0:D] = packed.astype(jnp.bfloat16)
    y_ref[:, D:128] = jnp.zeros((r, 128 - D), jnp.bfloat16)


def _tc_first_body(u_ref, i_ref, f_ref, wft_ref, bf_ref, w1_ref, s_ref,
                   x0_ref, y_ref):
    i = pl.program_id(0)
    sc = s_ref[...]
    item = i_ref[...] + jnp.dot(f_ref[...], wft_ref[...],
                                preferred_element_type=_f32) + bf_ref[...]
    x0 = jnp.where(i < UBLK, u_ref[...], item)
    x0_ref[...] = x0
    _store_y(jnp.dot(x0, w1_ref[...], preferred_element_type=_f32) * sc,
             y_ref)


def _tc_first(U, I, feat, WfT, bf2, W1, dis):
    return pl.pallas_call(
        _tc_first_body,
        grid=(GRID,),
        in_specs=[
            pl.BlockSpec((R, D), lambda i: (jnp.minimum(i, UBLK - 1), 0)),
            pl.BlockSpec((R, D), lambda i: (jnp.maximum(i - UBLK, 0), 0)),
            pl.BlockSpec((R, KG), lambda i: (jnp.maximum(i - UBLK, 0), 0)),
            pl.BlockSpec((KG, D), lambda i: (0, 0)),
            pl.BlockSpec((1, D), lambda i: (0, 0)),
            pl.BlockSpec((D, D), lambda i: (0, 0)),
            pl.BlockSpec((R, 1), lambda i: (i, 0)),
        ],
        out_specs=[
            pl.BlockSpec((R, D), lambda i: (i, 0)),
            pl.BlockSpec((R, 128), lambda i: (i, 0)),
        ],
        out_shape=[
            jax.ShapeDtypeStruct((NN, D), _f32),
            jax.ShapeDtypeStruct((NN, 128), jnp.bfloat16),
        ],
    )(U, I, feat, WfT, bf2, W1, dis)


def _merge_acc(acc_ref):
    return jnp.concatenate([acc_ref[0, :, 0:32], acc_ref[1, :, 0:32]],
                           axis=1)


def _tc_mid_body(acc_ref, s_ref, b_ref, w_ref, x_ref, y_ref):
    sc = s_ref[...]
    x = _merge_acc(acc_ref) * sc + b_ref[...]
    x_ref[...] = x
    _store_y(jnp.dot(x, w_ref[...], preferred_element_type=_f32) * sc, y_ref)


def _tc_mid(acc, dis, b2, W):
    return pl.pallas_call(
        _tc_mid_body,
        grid=(GRID,),
        in_specs=[
            pl.BlockSpec((2, R, 128), lambda i: (0, i, 0)),
            pl.BlockSpec((R, 1), lambda i: (i, 0)),
            pl.BlockSpec((1, D), lambda i: (0, 0)),
            pl.BlockSpec((D, D), lambda i: (0, 0)),
        ],
        out_specs=[
            pl.BlockSpec((R, D), lambda i: (i, 0)),
            pl.BlockSpec((R, 128), lambda i: (i, 0)),
        ],
        out_shape=[
            jax.ShapeDtypeStruct((NN, D), _f32),
            jax.ShapeDtypeStruct((NN, 128), jnp.bfloat16),
        ],
    )(acc, dis, b2, W)


def _tc_last_body(acc_ref, s_ref, b_ref, x0_ref, x1_ref, x2_ref, o_ref):
    sc = s_ref[...]
    x3 = _merge_acc(acc_ref) * sc + b_ref[...]
    o_ref[...] = (x0_ref[...] + x1_ref[...] + x2_ref[...] + x3) * 0.25


def _tc_last(acc, dis, b2, x0, x1, x2):
    blk = pl.BlockSpec((R, D), lambda i: (i, 0))
    return pl.pallas_call(
        _tc_last_body,
        grid=(GRID,),
        in_specs=[
            pl.BlockSpec((2, R, 128), lambda i: (0, i, 0)),
            pl.BlockSpec((R, 1), lambda i: (i, 0)),
            pl.BlockSpec((1, D), lambda i: (0, 0)),
            blk, blk, blk,
        ],
        out_specs=blk,
        out_shape=jax.ShapeDtypeStruct((NN, D), _f32),
    )(acc, dis, b2, x0, x1, x2)


def kernel(edge_index, item_features, U, I, Wf, bf, W1, b1, W2, b2, W3, b3):
    src = edge_index[0].astype(jnp.int32)
    dst = edge_index[1].astype(jnp.int32)
    pad = EP - E
    srcb = jnp.concatenate([src, jnp.zeros((pad,), jnp.int32)])
    dstb = jnp.concatenate([dst, jnp.full((pad,), TRASH, jnp.int32)])
    srcb = srcb.reshape(EROWS, 128)
    dstb = dstb.reshape(EROWS, 128)
    WfT = Wf.T
    bf2 = bf.reshape(1, D)

    deg2 = _sc_deg(dstb).reshape(2, NPASS, DROWS, 128)
    dis = _tc_dis(deg2)
    x0, y = _tc_first(U, I, item_features, WfT, bf2, W1, dis)
    acc = _sc_scatter(srcb, dstb, y)
    x1, y = _tc_mid(acc, dis, b1.reshape(1, D), W2)
    acc = _sc_scatter(srcb, dstb, y)
    x2, y = _tc_mid(acc, dis, b2.reshape(1, D), W3)
    acc = _sc_scatter(srcb, dstb, y)
    out = _tc_last(acc, dis, b3.reshape(1, D), x0, x1, x2)
    return out[:NU], out[NU:]


# speedup vs baseline: 3.5286x; 1.1183x over previous
import functools

import jax
import jax.numpy as jnp
from jax import lax
from jax.experimental import pallas as pl
from jax.experimental.pallas import tpu as pltpu
from jax.experimental.pallas import tpu_sc as plsc

NU = 25000
NI = 25000
NN = NU + NI
D = 64
KG = 128
E = 800000
EP = 802816
EROWS = EP // 128
TRASH = NN

SUB_ROWS = EROWS // 16
SC_BLK = 1
SC_OUTER = SUB_ROWS // SC_BLK
AROWS = 51200
AZSPAN = AROWS // 16

DEG_ROWS = EROWS // 32
DEG_OUTER = DEG_ROWS // 4
NPASS = 8
PASS_N = 6400
DROWS = 6528
DSPAN = DROWS // 16

R = 1000
GRID = NN // R
UBLK = NU // R
RD = 200
DGRID = NN // RD

_MESH = dict(core_axis_name="c", subcore_axis_name="s")
_f32 = jnp.float32


def _fill(ref, rows, value):
    zero16 = jnp.full((16,), value, _f32)

    def body(i, carry):
        for k in range(ref.shape[1] // 16):
            ref[i, pl.ds(k * 16, 16)] = zero16
        return carry

    lax.fori_loop(0, rows, body, 0)


def _deg_body(dstb, out, idx_d, idx_l, ones_v, zbuf, dbuf, dacc, sem):
    c = lax.axis_index("c")
    s = lax.axis_index("s")
    wid = s * 2 + c
    _fill(ones_v, 128, 1.0)
    _fill(zbuf, 128, 0.0)
    for p in range(NPASS):
        for r0, rn in ((0, 128), (128, 128), (256, 128), (384, 24)):
            pltpu.sync_copy(zbuf.at[pl.ds(0, rn)],
                            dacc.at[pl.ds(s * DSPAN + r0, rn)])
        plsc.subcore_barrier()

        def outer(i, carry):
            base = wid * DEG_ROWS + i * 4
            pltpu.sync_copy(dstb.at[pl.ds(base, 4)], idx_d)
            for r in range(4):
                for k in range(8):
                    v = idx_d[r, pl.ds(k * 16, 16)]
                    l = v - p * PASS_N
                    bad = (l < 0) | (l >= PASS_N)
                    idx_l[r, pl.ds(k * 16, 16)] = jnp.where(bad, PASS_N, l)
            for j in range(4):
                pltpu.sync_copy(ones_v, dacc.at[idx_l.at[j]], add=True)
            return carry

        lax.fori_loop(0, DEG_OUTER, outer, 0)
        plsc.subcore_barrier()
        for r0, rn in ((0, 128), (128, 128), (256, 128), (384, 24)):
            pltpu.sync_copy(dacc.at[pl.ds(s * DSPAN + r0, rn)],
                            dbuf.at[pl.ds(0, rn)])
            pltpu.sync_copy(
                dbuf.at[pl.ds(0, rn)],
                out.at[pl.ds((c * NPASS + p) * DROWS + s * DSPAN + r0, rn)])
        plsc.subcore_barrier()


@functools.partial(
    pl.kernel,
    out_type=jax.ShapeDtypeStruct((2 * NPASS * DROWS, 128), _f32),
    mesh=plsc.VectorSubcoreMesh(**_MESH),
    compiler_params=pltpu.CompilerParams(use_tc_tiling_on_sc=False),
    scratch_types=[
        pltpu.VMEM((4, 128), jnp.int32),
        pltpu.VMEM((4, 128), jnp.int32),
        pltpu.VMEM((128, 128), _f32),
        pltpu.VMEM((128, 128), _f32),
        pltpu.VMEM((128, 128), _f32),
        pltpu.VMEM_SHARED((DROWS, 128), _f32),
        pltpu.SemaphoreType.DMA,
    ],
)
def _sc_deg(dstb, out, idx_d, idx_l, ones_v, zbuf, dbuf, dacc, sem):
    _deg_body(dstb, out, idx_d, idx_l, ones_v, zbuf, dbuf, dacc, sem)


def _scatter_body(srcb, dstb, y, out, idx_s8, idx_d8,
                  rowsa, rowsb, rows32, zbuf, acc, sema, semb):
    c = lax.axis_index("c")
    s = lax.axis_index("s")
    _fill(zbuf, 128, 0.0)
    for k in range(AZSPAN // 128):
        pltpu.sync_copy(zbuf, acc.at[pl.ds(s * AZSPAN + k * 128, 128)])
    plsc.subcore_barrier()

    base0 = s * SUB_ROWS

    def _extract(rows_bf):
        def body(r, carry):
            v = rows_bf[r, pl.ds(c * 32, 32)]
            a, b = plsc.unpack(v, format=plsc.PackFormat.INTERLEAVED,
                               preferred_element_type=_f32)
            rows32[r, pl.ds(0, 16)] = a
            rows32[r, pl.ds(16, 16)] = b
            return carry

        lax.fori_loop(0, 128, body, 0)

    bufs = (rowsa, rowsb)
    sems = (sema, semb)
    NGROUP = SUB_ROWS // 8

    pltpu.sync_copy(srcb.at[pl.ds(base0, 8)], idx_s8)
    pltpu.sync_copy(dstb.at[pl.ds(base0, 8)], idx_d8)
    pltpu.async_copy(y.at[idx_s8.at[0]], rowsa, sema)

    def outer(g, carry):
        for j in range(8):
            buf, sem = bufs[j % 2], sems[j % 2]
            obuf, osem = bufs[1 - j % 2], sems[1 - j % 2]
            if j < 7:
                pltpu.async_copy(y.at[idx_s8.at[j + 1]], obuf, osem)
            pltpu.make_async_copy(y.at[idx_s8.at[j]], buf, sem).wait()
            _extract(buf)
            if j == 7:
                @pl.when(g < NGROUP - 1)
                def _stage_next_src():
                    pltpu.sync_copy(srcb.at[pl.ds(base0 + (g + 1) * 8, 8)],
                                    idx_s8)
                    pltpu.async_copy(y.at[idx_s8.at[0]], obuf, osem)

                pltpu.sync_copy(rows32, acc.at[idx_d8.at[j]], add=True)

                @pl.when(g < NGROUP - 1)
                def _stage_next_dst():
                    pltpu.sync_copy(dstb.at[pl.ds(base0 + (g + 1) * 8, 8)],
                                    idx_d8)
            else:
                pltpu.sync_copy(rows32, acc.at[idx_d8.at[j]], add=True)
        return carry

    lax.fori_loop(0, NGROUP, outer, 0)
    plsc.subcore_barrier()
    for k in range(25):
        r0 = s * AZSPAN + k * 128
        pltpu.sync_copy(acc.at[pl.ds(r0, 128)], rows32)
        pltpu.sync_copy(rows32,
                        out.at[c, pl.ds(r0, 128), pl.ds(0, 32)])


@functools.partial(
    pl.kernel,
    out_type=jax.ShapeDtypeStruct((2, AROWS, 128), _f32),
    mesh=plsc.VectorSubcoreMesh(**_MESH),
    compiler_params=pltpu.CompilerParams(use_tc_tiling_on_sc=False,
                                         needs_layout_passes=False),
    scratch_types=[
        pltpu.VMEM((8, 128), jnp.int32),
        pltpu.VMEM((8, 128), jnp.int32),
        pltpu.VMEM((128, 128), jnp.bfloat16),
        pltpu.VMEM((128, 128), jnp.bfloat16),
        pltpu.VMEM((128, 32), _f32),
        pltpu.VMEM((128, 32), _f32),
        pltpu.VMEM_SHARED((AROWS, 32), _f32),
        pltpu.SemaphoreType.DMA,
        pltpu.SemaphoreType.DMA,
    ],
)
def _sc_scatter(srcb, dstb, y, out, idx_s8, idx_d8, rowsa,
                rowsb, rows32, zbuf, acc, sema, semb):
    _scatter_body(srcb, dstb, y, out, idx_s8, idx_d8, rowsa,
                  rowsb, rows32, zbuf, acc, sema, semb)


def _dis_body(deg_ref, o_ref):
    dg = deg_ref[0, 0, :, 0:1] + deg_ref[1, 0, :, 0:1]
    o_ref[...] = jnp.where(dg > 0, lax.rsqrt(jnp.maximum(dg, 1e-12)), 0.0)


def _tc_dis(deg2):
    return pl.pallas_call(
        _dis_body,
        grid=(DGRID,),
        in_specs=[
            pl.BlockSpec((2, 1, RD, 128),
                         lambda i: (0, i // (PASS_N // RD),
                                    i % (PASS_N // RD), 0)),
        ],
        out_specs=pl.BlockSpec((RD, 1), lambda i: (i, 0)),
        out_shape=jax.ShapeDtypeStruct((NN, 1), _f32),
    )(deg2)


def _dis_block(deg_ref):
    dg = deg_ref[0, 0, :, 0:1] + deg_ref[1, 0, :, 0:1]
    return jnp.where(dg > 0, lax.rsqrt(jnp.maximum(dg, 1e-12)), 0.0)


def _store_y(y, y_ref):
    r = y.shape[0]
    packed = jnp.transpose(y.reshape(r, 2, 2, 16), (0, 1, 3, 2)).reshape(r, D)
    y_ref[:, 0:D] = packed.astype(jnp.bfloat16)
    y_ref[:, D:128] = jnp.zeros((r, 128 - D), jnp.bfloat16)


def _tc_first_body(u_ref, i_ref, f_ref, wft_ref, bf_ref, w1_ref, s_ref,
                   x0_ref, y_ref):
    i = pl.program_id(0)
    sc = s_ref[...]
    item = i_ref[...] + jnp.dot(f_ref[...], wft_ref[...],
                                preferred_element_type=_f32) + bf_ref[...]
    x0 = jnp.where(i < UBLK, u_ref[...], item)
    x0_ref[...] = x0
    _store_y(jnp.dot(x0, w1_ref[...], preferred_element_type=_f32) * sc,
             y_ref)


def _tc_first(U, I, feat, WfT, bf2, W1, dis):
    return pl.pallas_call(
        _tc_first_body,
        grid=(GRID,),
        in_specs=[
            pl.BlockSpec((R, D), lambda i: (jnp.minimum(i, UBLK - 1), 0)),
            pl.BlockSpec((R, D), lambda i: (jnp.maximum(i - UBLK, 0), 0)),
            pl.BlockSpec((R, KG), lambda i: (jnp.maximum(i - UBLK, 0), 0)),
            pl.BlockSpec((KG, D), lambda i: (0, 0)),
            pl.BlockSpec((1, D), lambda i: (0, 0)),
            pl.BlockSpec((D, D), lambda i: (0, 0)),
            pl.BlockSpec((R, 1), lambda i: (i, 0)),
        ],
        out_specs=[
            pl.BlockSpec((R, D), lambda i: (i, 0)),
            pl.BlockSpec((R, 128), lambda i: (i, 0)),
        ],
        out_shape=[
            jax.ShapeDtypeStruct((NN, D), _f32),
            jax.ShapeDtypeStruct((NN, 128), jnp.bfloat16),
        ],
    )(U, I, feat, WfT, bf2, W1, dis)


def _merge_acc(acc_ref):
    return jnp.concatenate([acc_ref[0, :, 0:32], acc_ref[1, :, 0:32]],
                           axis=1)


def _tc_mid_body(acc_ref, s_ref, b_ref, w_ref, x_ref, y_ref):
    sc = s_ref[...]
    x = _merge_acc(acc_ref) * sc + b_ref[...]
    x_ref[...] = x
    _store_y(jnp.dot(x, w_ref[...], preferred_element_type=_f32) * sc, y_ref)


def _tc_mid(acc, dis, b2, W):
    return pl.pallas_call(
        _tc_mid_body,
        grid=(GRID,),
        in_specs=[
            pl.BlockSpec((2, R, 128), lambda i: (0, i, 0)),
            pl.BlockSpec((R, 1), lambda i: (i, 0)),
            pl.BlockSpec((1, D), lambda i: (0, 0)),
            pl.BlockSpec((D, D), lambda i: (0, 0)),
        ],
        out_specs=[
            pl.BlockSpec((R, D), lambda i: (i, 0)),
            pl.BlockSpec((R, 128), lambda i: (i, 0)),
        ],
        out_shape=[
            jax.ShapeDtypeStruct((NN, D), _f32),
            jax.ShapeDtypeStruct((NN, 128), jnp.bfloat16),
        ],
    )(acc, dis, b2, W)


def _tc_last_body(acc_ref, s_ref, b_ref, x0_ref, x1_ref, x2_ref, o_ref):
    sc = s_ref[...]
    x3 = _merge_acc(acc_ref) * sc + b_ref[...]
    o_ref[...] = (x0_ref[...] + x1_ref[...] + x2_ref[...] + x3) * 0.25


def _tc_last(acc, dis, b2, x0, x1, x2):
    blk = pl.BlockSpec((R, D), lambda i: (i, 0))
    return pl.pallas_call(
        _tc_last_body,
        grid=(GRID,),
        in_specs=[
            pl.BlockSpec((2, R, 128), lambda i: (0, i, 0)),
            pl.BlockSpec((R, 1), lambda i: (i, 0)),
            pl.BlockSpec((1, D), lambda i: (0, 0)),
            blk, blk, blk,
        ],
        out_specs=blk,
        out_shape=jax.ShapeDtypeStruct((NN, D), _f32),
    )(acc, dis, b2, x0, x1, x2)


def kernel(edge_index, item_features, U, I, Wf, bf, W1, b1, W2, b2, W3, b3):
    src = edge_index[0].astype(jnp.int32)
    dst = edge_index[1].astype(jnp.int32)
    pad = EP - E
    srcb = jnp.concatenate([src, jnp.zeros((pad,), jnp.int32)])
    dstb = jnp.concatenate([dst, jnp.full((pad,), TRASH, jnp.int32)])
    srcb = srcb.reshape(EROWS, 128)
    dstb = dstb.reshape(EROWS, 128)
    WfT = Wf.T
    bf2 = bf.reshape(1, D)

    deg2 = _sc_deg(dstb).reshape(2, NPASS, DROWS, 128)
    dis = _tc_dis(deg2)
    x0, y = _tc_first(U, I, item_features, WfT, bf2, W1, dis)
    acc = _sc_scatter(srcb, dstb, y)
    x1, y = _tc_mid(acc, dis, b1.reshape(1, D), W2)
    acc = _sc_scatter(srcb, dstb, y)
    x2, y = _tc_mid(acc, dis, b2.reshape(1, D), W3)
    acc = _sc_scatter(srcb, dstb, y)
    out = _tc_last(acc, dis, b3.reshape(1, D), x0, x1, x2)
    return out[:NU], out[NU:]
